# Initial kernel scaffold; baseline (speedup 1.0000x reference)
import jax
import jax.numpy as jnp
from jax.experimental import pallas as pl


def kernel(patches):
    raise NotImplementedError("write your pallas kernel here")


# speedup vs baseline: 1.2014x; 1.2014x over previous
import functools

import jax
import jax.numpy as jnp
from jax import lax
from jax.experimental import pallas as pl
from jax.experimental.pallas import tpu as pltpu
from jax.experimental.pallas import tpu_sc as plsc

GROUP_N = 32

_NC = 2
_NS = 16
_NW = _NC * _NS
_K = 128


def _permutation_indexes(T, group_n):
    key = jax.random.key(42)
    keys = jax.random.split(key, group_n)
    fwd = jnp.stack([jax.random.permutation(k, T) for k in keys], axis=0).astype(jnp.int64)
    bwd = jnp.argsort(fwd, axis=1).astype(jnp.int64)
    return fwd, bwd


def _build_gather(rows, C, nchunk):
    mesh = plsc.VectorSubcoreMesh(core_axis_name="c", subcore_axis_name="s")

    @functools.partial(
        pl.kernel,
        mesh=mesh,
        out_type=jax.ShapeDtypeStruct((rows, C), jnp.float32),
        scratch_types=[
            pltpu.VMEM((nchunk, _K), jnp.int32),
            pltpu.VMEM((_K, C), jnp.float32),
            pltpu.SemaphoreType.DMA,
        ],
    )
    def gather_kernel(table_hbm, idx_hbm, out_hbm, idx_v, buf, sem):
        wid = lax.axis_index("s") * _NC + lax.axis_index("c")
        base = wid * (nchunk * _K)
        pltpu.sync_copy(idx_hbm.at[wid], idx_v)

        def body(j, carry):
            pltpu.async_copy(table_hbm.at[idx_v.at[j]], buf, sem).wait()
            pltpu.sync_copy(buf, out_hbm.at[pl.ds(base + j * _K, _K)])
            return carry

        lax.fori_loop(0, nchunk, body, 0)

    return gather_kernel


def kernel(patches):
    B, T, C = patches.shape
    group_n = B // GROUP_N
    fwd, bwd = _permutation_indexes(T, group_n)
    forward_indexes = jnp.repeat(fwd, GROUP_N, axis=0)
    backward_indexes = jnp.repeat(bwd, GROUP_N, axis=0)

    rows = B * T
    per_w = rows // _NW
    nchunk = per_w // _K
    assert per_w * _NW == rows and nchunk * _K == per_w

    gidx = (
        jnp.arange(B, dtype=jnp.int32)[:, None] * T
        + forward_indexes.astype(jnp.int32)
    ).reshape(_NW, nchunk, _K)
    table = patches.reshape(rows, C)

    shuffled = _build_gather(rows, C, nchunk)(table, gidx)
    return shuffled.reshape(B, T, C), forward_indexes, backward_indexes


# speedup vs baseline: 1.5235x; 1.2681x over previous
import functools

import jax
import jax.numpy as jnp
from jax import lax
from jax.experimental import pallas as pl
from jax.experimental.pallas import tpu as pltpu
from jax.experimental.pallas import tpu_sc as plsc

GROUP_N = 32

_NC = 2
_NS = 16
_NW = _NC * _NS
_K = 128


def _permutation_indexes(T, group_n):
    key = jax.random.key(42)
    keys = jax.random.split(key, group_n)
    fwd = jnp.stack([jax.random.permutation(k, T) for k in keys], axis=0).astype(jnp.int64)
    bwd = jnp.argsort(fwd, axis=1).astype(jnp.int64)
    return fwd, bwd


def _build_gather(rows, C, nchunk):
    mesh = plsc.VectorSubcoreMesh(core_axis_name="c", subcore_axis_name="s")

    @functools.partial(
        pl.kernel,
        mesh=mesh,
        out_type=jax.ShapeDtypeStruct((rows, C), jnp.float32),
        scratch_types=[
            pltpu.VMEM((nchunk, _K), jnp.int32),
            pltpu.VMEM((_K, C), jnp.float32),
            pltpu.VMEM((_K, C), jnp.float32),
            pltpu.SemaphoreType.DMA,
            pltpu.SemaphoreType.DMA,
        ],
    )
    def gather_kernel(table_hbm, idx_hbm, out_hbm, idx_v, buf0, buf1, sem0, sem1):
        wid = lax.axis_index("s") * _NC + lax.axis_index("c")
        base = wid * (nchunk * _K)
        pltpu.sync_copy(idx_hbm.at[wid], idx_v)

        def start_gather(j, buf, sem):
            pltpu.async_copy(table_hbm.at[idx_v.at[j]], buf, sem)

        def wait_gather(buf, sem):
            pltpu.make_async_copy(table_hbm.at[pl.ds(0, _K)], buf, sem).wait()

        start_gather(0, buf0, sem0)

        def body(jj, carry):
            j0 = jj * 2
            start_gather(j0 + 1, buf1, sem1)
            wait_gather(buf0, sem0)
            pltpu.sync_copy(buf0, out_hbm.at[pl.ds(base + j0 * _K, _K)])

            @pl.when(jj < nchunk // 2 - 1)
            def _():
                start_gather(j0 + 2, buf0, sem0)

            wait_gather(buf1, sem1)
            pltpu.sync_copy(buf1, out_hbm.at[pl.ds(base + (j0 + 1) * _K, _K)])
            return carry

        lax.fori_loop(0, nchunk // 2, body, 0)

    return gather_kernel


def kernel(patches):
    B, T, C = patches.shape
    group_n = B // GROUP_N
    fwd, bwd = _permutation_indexes(T, group_n)
    forward_indexes = jnp.repeat(fwd, GROUP_N, axis=0)
    backward_indexes = jnp.repeat(bwd, GROUP_N, axis=0)

    rows = B * T
    per_w = rows // _NW
    nchunk = per_w // _K
    assert per_w * _NW == rows and nchunk * _K == per_w

    gidx = (
        jnp.arange(B, dtype=jnp.int32)[:, None] * T
        + forward_indexes.astype(jnp.int32)
    ).reshape(_NW, nchunk, _K)
    table = patches.reshape(rows, C)

    shuffled = _build_gather(rows, C, nchunk)(table, gidx)
    return shuffled.reshape(B, T, C), forward_indexes, backward_indexes


# speedup vs baseline: 2.2902x; 1.5033x over previous
import functools

import jax
import jax.numpy as jnp
import numpy as np
from jax import lax
from jax.experimental import pallas as pl
from jax.experimental.pallas import tpu as pltpu
from jax.experimental.pallas import tpu_sc as plsc

GROUP_N = 32

_NC = 2
_NS = 16
_NW = _NC * _NS
_K = 128


def _permutation_indexes(T, group_n):
    key = jax.random.key(42)
    keys = jax.random.split(key, group_n)
    fwd = jnp.stack([jax.random.permutation(k, T) for k in keys], axis=0).astype(jnp.int64)
    bwd = jnp.argsort(fwd, axis=1).astype(jnp.int64)
    return fwd, bwd


@functools.lru_cache(maxsize=None)
def _index_constants(B, T):
    with jax.ensure_compile_time_eval():
        fwd, bwd = _permutation_indexes(T, B // GROUP_N)
    fwd_rep = np.repeat(np.asarray(fwd), GROUP_N, axis=0)
    bwd_rep = np.repeat(np.asarray(bwd), GROUP_N, axis=0)
    gidx = (np.arange(B, dtype=np.int32)[:, None] * T
            + fwd_rep.astype(np.int32)).reshape(-1)
    return fwd_rep, bwd_rep, gidx


def _build_gather(rows, C, nchunk):
    mesh = plsc.VectorSubcoreMesh(core_axis_name="c", subcore_axis_name="s")

    @functools.partial(
        pl.kernel,
        mesh=mesh,
        out_type=jax.ShapeDtypeStruct((rows, C), jnp.float32),
        scratch_types=[
            pltpu.VMEM((nchunk, _K), jnp.int32),
            pltpu.VMEM((_K, C), jnp.float32),
            pltpu.VMEM((_K, C), jnp.float32),
            pltpu.SemaphoreType.DMA,
            pltpu.SemaphoreType.DMA,
        ],
    )
    def gather_kernel(table_hbm, idx_hbm, out_hbm, idx_v, buf0, buf1, sem0, sem1):
        wid = lax.axis_index("s") * _NC + lax.axis_index("c")
        base = wid * (nchunk * _K)
        pltpu.sync_copy(idx_hbm.at[wid], idx_v)

        def start_gather(j, buf, sem):
            pltpu.async_copy(table_hbm.at[idx_v.at[j]], buf, sem)

        def wait_gather(buf, sem):
            pltpu.make_async_copy(table_hbm.at[pl.ds(0, _K)], buf, sem).wait()

        start_gather(0, buf0, sem0)

        def body(jj, carry):
            j0 = jj * 2
            start_gather(j0 + 1, buf1, sem1)
            wait_gather(buf0, sem0)
            pltpu.sync_copy(buf0, out_hbm.at[pl.ds(base + j0 * _K, _K)])

            @pl.when(jj < nchunk // 2 - 1)
            def _():
                start_gather(j0 + 2, buf0, sem0)

            wait_gather(buf1, sem1)
            pltpu.sync_copy(buf1, out_hbm.at[pl.ds(base + (j0 + 1) * _K, _K)])
            return carry

        lax.fori_loop(0, nchunk // 2, body, 0)

    return gather_kernel


def kernel(patches):
    B, T, C = patches.shape
    fwd_rep, bwd_rep, gidx = _index_constants(B, T)

    rows = B * T
    per_w = rows // _NW
    nchunk = per_w // _K
    assert per_w * _NW == rows and nchunk * _K == per_w

    table = patches.reshape(rows, C)
    shuffled = _build_gather(rows, C, nchunk)(
        table, jnp.asarray(gidx.reshape(_NW, nchunk, _K)))
    return (shuffled.reshape(B, T, C),
            jnp.asarray(fwd_rep), jnp.asarray(bwd_rep))
